# baseline (speedup 1.0000x reference)
import functools

import jax
import jax.numpy as jnp
from jax import lax
from jax.experimental import pallas as pl
from jax.experimental.pallas import tpu as pltpu
from jax.experimental.pallas import tpu_sc as plsc

N = 10000
Dm = 256
E = 160000
NR = 5
HW = Dm // 2
SUB = 16
B = 80
NCH = (E // SUB) // B
NBLK = NR * SUB * NCH
RB = 2000

_f32 = jnp.float32
_i32 = jnp.int32


def _dense_body(nb_ref, st_ref, w_ref, awn_ref, aws_ref,
                f0_ref, f1_ref, v_ref, u_ref):
    nb = nb_ref[...]
    dn = (((1,), (1,)), ((), ()))
    f0_ref[...] = lax.dot_general(nb, w_ref[:HW, :], dn,
                                  preferred_element_type=_f32)
    f1_ref[...] = lax.dot_general(nb, w_ref[HW:, :], dn,
                                  preferred_element_type=_f32)
    v_ref[...] = jnp.dot(nb, awn_ref[...], preferred_element_type=_f32)
    u_ref[...] = jnp.dot(st_ref[...], aws_ref[...], preferred_element_type=_f32)


_dense = pl.pallas_call(
    _dense_body,
    grid=(N // RB,),
    in_specs=[
        pl.BlockSpec((RB, Dm), lambda i: (i, 0)),
        pl.BlockSpec((RB, Dm), lambda i: (i, 0)),
        pl.BlockSpec((Dm, Dm), lambda i: (0, 0)),
        pl.BlockSpec((Dm, 1), lambda i: (0, 0)),
        pl.BlockSpec((Dm, 1), lambda i: (0, 0)),
    ],
    out_specs=[
        pl.BlockSpec((RB, HW), lambda i: (i, 0)),
        pl.BlockSpec((RB, HW), lambda i: (i, 0)),
        pl.BlockSpec((RB, 1), lambda i: (i, 0)),
        pl.BlockSpec((RB, 1), lambda i: (i, 0)),
    ],
    out_shape=[
        jax.ShapeDtypeStruct((N, HW), _f32),
        jax.ShapeDtypeStruct((N, HW), _f32),
        jax.ShapeDtypeStruct((N, 1), _f32),
        jax.ShapeDtypeStruct((N, 1), _f32),
    ],
)


def _shift_body(u0_ref, v_ref, u_ref):
    m = jnp.max(u0_ref[...]) + jnp.max(v_ref[...])
    u_ref[...] = u0_ref[...] - m


_shift = pl.pallas_call(
    _shift_body,
    out_shape=jax.ShapeDtypeStruct((N, 1), _f32),
)


def _combine_body(ce_ref, a1_ref, s1_ref, a2_ref, s2_ref,
                  ie_ref, a3_ref, s3_ref, a4_ref, s4_ref,
                  se_ref, a5_ref, s5_ref,
                  co_ref, io_ref, so_ref):
    tiny = jnp.float32(1e-30)
    s1 = jnp.sum(s1_ref[...], axis=1, keepdims=True)
    s2 = jnp.sum(s2_ref[...], axis=1, keepdims=True)
    s3 = jnp.sum(s3_ref[...], axis=1, keepdims=True)
    s4 = jnp.sum(s4_ref[...], axis=1, keepdims=True)
    s5 = jnp.sum(s5_ref[...], axis=1, keepdims=True)
    r1 = a1_ref[...] / jnp.maximum(s1, tiny)
    r2 = a2_ref[...] / jnp.maximum(s2, tiny)
    r3 = a3_ref[...] / jnp.maximum(s3, tiny)
    r4 = a4_ref[...] / jnp.maximum(s4, tiny)
    r5 = a5_ref[...] / jnp.maximum(s5, tiny)
    ce = ce_ref[...]
    co_ref[...] = jnp.where(s1 > 0, ce + r1 + r2, ce)
    io_ref[...] = ie_ref[...] + r3 + r4
    so_ref[...] = se_ref[...] + r5


_mat_spec = pl.BlockSpec((RB, Dm), lambda i: (i, 0))
_col_spec = pl.BlockSpec((RB, SUB), lambda i: (i, 0))

_combine = pl.pallas_call(
    _combine_body,
    grid=(N // RB,),
    in_specs=[_mat_spec, _mat_spec, _col_spec, _mat_spec, _col_spec,
              _mat_spec, _mat_spec, _col_spec, _mat_spec, _col_spec,
              _mat_spec, _mat_spec, _col_spec],
    out_specs=[_mat_spec, _mat_spec, _mat_spec],
    out_shape=[
        jax.ShapeDtypeStruct((N, Dm), _f32),
        jax.ShapeDtypeStruct((N, Dm), _f32),
        jax.ShapeDtypeStruct((N, Dm), _f32),
    ],
)


_sc_mesh = plsc.VectorSubcoreMesh(core_axis_name="c", subcore_axis_name="s",
                                  num_cores=2, num_subcores=SUB)
_sc_params = pltpu.CompilerParams(use_tc_tiling_on_sc=False,
                                  needs_layout_passes=False)


@functools.partial(
    pl.kernel,
    out_type=[jax.ShapeDtypeStruct((NBLK, 3, B), _i32),
              jax.ShapeDtypeStruct((NR * SUB, N), _f32)],
    mesh=_sc_mesh,
    compiler_params=_sc_params,
    scratch_types=[
        pltpu.VMEM((NCH, B), _i32),
        pltpu.VMEM((NCH, B), _i32),
        pltpu.VMEM((N,), _f32),
        pltpu.VMEM((N,), _f32),
        pltpu.VMEM((NCH, 3, B), _i32),
        pltpu.VMEM((N,), _f32),
    ],
)
def _sc_pw(u5, v5, ds, sr, edata, spart, dst_v, src_v, u_v, v_v, ostg, s_acc):
    c = lax.axis_index("c")
    s = lax.axis_index("s")
    wid = s * 2 + c

    for t in range((NR * SUB + 31) // 32):
        blk = wid + t * 32

        @pl.when(blk < NR * SUB)
        def _():
            r = blk // SUB
            pltpu.sync_copy(ds.at[blk], dst_v)
            pltpu.sync_copy(sr.at[blk], src_v)
            pltpu.sync_copy(u5.at[r], u_v)
            pltpu.sync_copy(v5.at[r], v_v)
            roff = r * N

            def sz_body(j, carry):
                s_acc[pl.ds(j * 16, 16)] = jnp.zeros((16,), _f32)
                return carry

            lax.fori_loop(0, N // 16, sz_body, 0)

            def p_body(j, carry):
                for k in range(B // 16):
                    sl = pl.ds(k * 16, 16)
                    d16 = dst_v[j, sl]
                    s16 = src_v[j, sl]
                    ud = plsc.load_gather(u_v, [d16])
                    vs = plsc.load_gather(v_v, [s16])
                    p16 = jnp.exp(ud + vs)
                    plsc.addupdate_scatter(s_acc, [d16], p16)
                    ostg[j, 0, sl] = d16
                    ostg[j, 1, sl] = s16 + roff
                    ostg[j, 2, sl] = plsc.bitcast(p16, _i32)
                return carry

            lax.fori_loop(0, NCH, p_body, 0)
            pltpu.sync_copy(ostg, edata.at[pl.ds(blk * NCH, NCH)])
            pltpu.sync_copy(s_acc, spart.at[blk])


@functools.partial(
    pl.kernel,
    out_type=jax.ShapeDtypeStruct((NR * 2 * N, HW), _f32),
    mesh=_sc_mesh,
    compiler_params=_sc_params,
    scratch_types=[
        pltpu.VMEM((3, B), _i32),
        pltpu.VMEM((3, B), _i32),
        pltpu.VMEM((3, B), _i32),
        pltpu.VMEM((3, B), _i32),
        pltpu.VMEM((B, HW), _f32),
        pltpu.VMEM((B, HW), _f32),
        pltpu.VMEM((B, HW), _f32),
        pltpu.VMEM((B, HW), _f32),
        pltpu.VMEM_SHARED((N, HW), _f32),
        pltpu.SemaphoreType.DMA,
        pltpu.SemaphoreType.DMA,
        pltpu.SemaphoreType.DMA,
        pltpu.SemaphoreType.DMA,
        pltpu.SemaphoreType.DMA,
        pltpu.SemaphoreType.DMA,
        pltpu.SemaphoreType.DMA,
        pltpu.SemaphoreType.DMA,
        pltpu.SemaphoreType.DMA,
        pltpu.SemaphoreType.DMA,
        pltpu.SemaphoreType.DMA,
        pltpu.SemaphoreType.DMA,
    ],
)
def _sc_acc(ft0, ft1, edata, out,
            eb0, eb1, eb2, eb3, rw0, rw1, rw2, rw3, acc,
            es0, es1, es2, es3, gs0, gs1, gs2, gs3, ss0, ss1, ss2, ss3):
    c = lax.axis_index("c")
    s = lax.axis_index("s")
    ebuf = (eb0, eb1, eb2, eb3)
    esem = (es0, es1, es2, es3)
    rows = (rw0, rw1, rw2, rw3)
    gsem = (gs0, gs1, gs2, gs3)
    ssem = (ss0, ss1, ss2, ss3)
    nblk_n = N // B

    def rel_body(r, carry):
        base = (r * SUB + s) * NCH

        def z_body(rr, carry1):
            for k in range(HW // 16):
                rw0[rr, pl.ds(k * 16, 16)] = jnp.zeros((16,), _f32)
            return carry1

        lax.fori_loop(0, B, z_body, 0)
        for t in range((nblk_n + SUB - 1) // SUB):
            blk = s + t * SUB

            @pl.when(blk < nblk_n)
            def _():
                pltpu.sync_copy(rw0, acc.at[pl.ds(blk * B, B)])
        plsc.subcore_barrier()

        pltpu.async_copy(edata.at[base], eb0, es0)
        pltpu.async_copy(edata.at[base + 1], eb1, es1)

        def m_body(m, carry2):
            for q in range(4):
                j = 4 * m + q
                q2 = (q + 2) % 4
                qp = (q + 3) % 4

                @pl.when(j + 2 < NCH)
                def _():
                    @pl.when(j >= 2)
                    def _():
                        pltpu.make_async_copy(
                            rows[q2], acc.at[ebuf[q2].at[0]],
                            ssem[q2]).wait()
                    pltpu.async_copy(edata.at[base + j + 2],
                                     ebuf[q2], esem[q2])

                @pl.when(j < NCH)
                def _():
                    pltpu.make_async_copy(
                        edata.at[base], ebuf[q], esem[q]).wait()

                    @pl.when(c == 0)
                    def _():
                        pltpu.async_copy(ft0.at[ebuf[q].at[1]], rows[q],
                                         gsem[q])

                    @pl.when(c == 1)
                    def _():
                        pltpu.async_copy(ft1.at[ebuf[q].at[1]], rows[q],
                                         gsem[q])

                @pl.when((j >= 1) & (j <= NCH))
                def _():
                    pltpu.make_async_copy(
                        ft0.at[ebuf[qp].at[1]], rows[qp], gsem[qp]).wait()

                    def scale_grp(g, carry3):
                        pvec = plsc.bitcast(
                            ebuf[qp][2, pl.ds(g * 16, 16)], _f32)
                        for rr in range(16):
                            rowi = g * 16 + rr
                            spl = pvec[rr]
                            for k in range(HW // 16):
                                sl = pl.ds(k * 16, 16)
                                rows[qp][rowi, sl] = (
                                    rows[qp][rowi, sl] * spl)
                        return carry3

                    lax.fori_loop(0, B // 16, scale_grp, 0)
                    pltpu.async_copy(rows[qp], acc.at[ebuf[qp].at[0]],
                                     ssem[qp], add=True)
            return carry2

        lax.fori_loop(0, (NCH + 2 + 3) // 4, m_body, 0)
        for q in range(4):
            pltpu.make_async_copy(rows[q], acc.at[eb0.at[0]],
                                  ssem[q]).wait()
        plsc.subcore_barrier()

        for t in range((nblk_n + SUB - 1) // SUB):
            blk = s + t * SUB

            @pl.when(blk < nblk_n)
            def _():
                pltpu.sync_copy(acc.at[pl.ds(blk * B, B)], rw0)
                pltpu.sync_copy(
                    rw0, out.at[pl.ds((2 * r + c) * N + blk * B, B)])
        return carry

    lax.fori_loop(0, NR, rel_body, 0)


def kernel(concept_emb, item_emb, stu_emb, cc_edge_index, ic_edge_index,
           si_edge_index, Wc1, Wc2, Wi1, Wi2, Ws,
           ac1_w, ac1_b, ac2_w, ac2_b, ace1_w, ace1_b, ace2_w, ace2_b,
           ai1_w, ai1_b, ai2_w, ai2_b, aie1_w, aie1_b, aie2_w, aie2_b,
           asl_w, asl_b):
    ce, ie, se = concept_emb, item_emb, stu_emb
    cc, ic, si = cc_edge_index, ic_edge_index, si_edge_index

    rels = [
        (ce, ce, Wc1, ac1_w, cc[1], cc[0]),
        (ce, ie, Wc2, ac2_w, ic[0], ic[1]),
        (ie, ce, Wi1, ai1_w, ic[1], ic[0]),
        (ie, se, Wi2, ai2_w, si[0], si[1]),
        (se, ie, Ws, asl_w, si[1], si[0]),
    ]

    us, vs, f0s, f1s, dss, srs = [], [], [], [], [], []
    for self_t, nbr_t, W, aw, src, dst in rels:
        f0, f1, v, u0 = _dense(nbr_t, self_t, W,
                               aw[Dm:].reshape(Dm, 1), aw[:Dm].reshape(Dm, 1))
        u = _shift(u0, v)
        us.append(u.reshape(N))
        vs.append(v.reshape(N))
        f0s.append(f0)
        f1s.append(f1)
        dss.append(dst.reshape(SUB, NCH, B))
        srs.append(src.reshape(SUB, NCH, B))

    u5 = jnp.stack(us)
    v5 = jnp.stack(vs)
    ft0 = jnp.concatenate(f0s, axis=0)
    ft1 = jnp.concatenate(f1s, axis=0)
    ds = jnp.concatenate(dss, axis=0)
    sr = jnp.concatenate(srs, axis=0)

    edata, spart = _sc_pw(u5, v5, ds, sr)
    o = _sc_acc(ft0, ft1, edata)

    o = o.reshape(NR, 2, N, HW)
    a = [jnp.concatenate([o[r, 0], o[r, 1]], axis=1) for r in range(NR)]
    sp = spart.reshape(NR, SUB, N).transpose(0, 2, 1)

    conc_out, item_out, stu_out = _combine(
        ce, a[0], sp[0], a[1], sp[1],
        ie, a[2], sp[2], a[3], sp[3],
        se, a[4], sp[4])
    return (conc_out, item_out, stu_out)


# speedup vs baseline: 1.0527x; 1.0527x over previous
import functools

import jax
import jax.numpy as jnp
from jax import lax
from jax.experimental import pallas as pl
from jax.experimental.pallas import tpu as pltpu
from jax.experimental.pallas import tpu_sc as plsc

N = 10000
Dm = 256
E = 160000
NR = 5
HW = Dm // 2
SUB = 16
B = 80
NCH = (E // SUB) // B
NBLK = NR * SUB * NCH
RB = 2000

_f32 = jnp.float32
_i32 = jnp.int32


def _dense_body(nb_ref, st_ref, w_ref, awn_ref, aws_ref,
                f0_ref, f1_ref, v_ref, u_ref):
    nb = nb_ref[...]
    dn = (((1,), (1,)), ((), ()))
    f0_ref[...] = lax.dot_general(nb, w_ref[:HW, :], dn,
                                  preferred_element_type=_f32)
    f1_ref[...] = lax.dot_general(nb, w_ref[HW:, :], dn,
                                  preferred_element_type=_f32)
    v_ref[...] = jnp.dot(nb, awn_ref[...], preferred_element_type=_f32)
    u_ref[...] = jnp.dot(st_ref[...], aws_ref[...], preferred_element_type=_f32)


_dense = pl.pallas_call(
    _dense_body,
    grid=(N // RB,),
    in_specs=[
        pl.BlockSpec((RB, Dm), lambda i: (i, 0)),
        pl.BlockSpec((RB, Dm), lambda i: (i, 0)),
        pl.BlockSpec((Dm, Dm), lambda i: (0, 0)),
        pl.BlockSpec((Dm, 1), lambda i: (0, 0)),
        pl.BlockSpec((Dm, 1), lambda i: (0, 0)),
    ],
    out_specs=[
        pl.BlockSpec((RB, HW), lambda i: (i, 0)),
        pl.BlockSpec((RB, HW), lambda i: (i, 0)),
        pl.BlockSpec((RB, 1), lambda i: (i, 0)),
        pl.BlockSpec((RB, 1), lambda i: (i, 0)),
    ],
    out_shape=[
        jax.ShapeDtypeStruct((N, HW), _f32),
        jax.ShapeDtypeStruct((N, HW), _f32),
        jax.ShapeDtypeStruct((N, 1), _f32),
        jax.ShapeDtypeStruct((N, 1), _f32),
    ],
)


def _shift_body(u0_ref, v_ref, u_ref):
    m = jnp.max(u0_ref[...]) + jnp.max(v_ref[...])
    u_ref[...] = u0_ref[...] - m


_shift = pl.pallas_call(
    _shift_body,
    grid=(NR,),
    in_specs=[pl.BlockSpec((1, 1, N), lambda i: (i, 0, 0)),
              pl.BlockSpec((1, 1, N), lambda i: (i, 0, 0))],
    out_specs=pl.BlockSpec((1, 1, N), lambda i: (i, 0, 0)),
    out_shape=jax.ShapeDtypeStruct((NR, 1, N), _f32),
)


def _combine_body(ce_ref, a1l_ref, a1r_ref, s1_ref, a2l_ref, a2r_ref, s2_ref,
                  ie_ref, a3l_ref, a3r_ref, s3_ref, a4l_ref, a4r_ref, s4_ref,
                  se_ref, a5l_ref, a5r_ref, s5_ref,
                  co_ref, io_ref, so_ref):
    tiny = jnp.float32(1e-30)

    def term(al_ref, ar_ref, s_ref):
        sden = jnp.sum(s_ref[...], axis=1, keepdims=True)
        q = 1.0 / jnp.maximum(sden, tiny)
        return sden, jnp.concatenate(
            [al_ref[...] * q, ar_ref[...] * q], axis=1)

    s1, r1 = term(a1l_ref, a1r_ref, s1_ref)
    _, r2 = term(a2l_ref, a2r_ref, s2_ref)
    _, r3 = term(a3l_ref, a3r_ref, s3_ref)
    _, r4 = term(a4l_ref, a4r_ref, s4_ref)
    _, r5 = term(a5l_ref, a5r_ref, s5_ref)
    ce = ce_ref[...]
    co_ref[...] = jnp.where(s1 > 0, ce + r1 + r2, ce)
    io_ref[...] = ie_ref[...] + r3 + r4
    so_ref[...] = se_ref[...] + r5


_mat_spec = pl.BlockSpec((RB, Dm), lambda i: (i, 0))
_half_spec = pl.BlockSpec((RB, HW), lambda i: (i, 0))
_col_spec = pl.BlockSpec((RB, SUB), lambda i: (i, 0))

_combine = pl.pallas_call(
    _combine_body,
    grid=(N // RB,),
    in_specs=[_mat_spec, _half_spec, _half_spec, _col_spec,
              _half_spec, _half_spec, _col_spec,
              _mat_spec, _half_spec, _half_spec, _col_spec,
              _half_spec, _half_spec, _col_spec,
              _mat_spec, _half_spec, _half_spec, _col_spec],
    out_specs=[_mat_spec, _mat_spec, _mat_spec],
    out_shape=[
        jax.ShapeDtypeStruct((N, Dm), _f32),
        jax.ShapeDtypeStruct((N, Dm), _f32),
        jax.ShapeDtypeStruct((N, Dm), _f32),
    ],
)


_sc_mesh = plsc.VectorSubcoreMesh(core_axis_name="c", subcore_axis_name="s",
                                  num_cores=2, num_subcores=SUB)
_sc_params = pltpu.CompilerParams(use_tc_tiling_on_sc=False,
                                  needs_layout_passes=False)


@functools.partial(
    pl.kernel,
    out_type=[jax.ShapeDtypeStruct((NBLK, 3, B), _i32),
              jax.ShapeDtypeStruct((NR * SUB, N), _f32)],
    mesh=_sc_mesh,
    compiler_params=_sc_params,
    scratch_types=[
        pltpu.VMEM((NCH, B), _i32),
        pltpu.VMEM((NCH, B), _i32),
        pltpu.VMEM((N,), _f32),
        pltpu.VMEM((N,), _f32),
        pltpu.VMEM((NCH, 3, B), _i32),
        pltpu.VMEM((N,), _f32),
    ],
)
def _sc_pw(u5, v5, ds, sr, edata, spart, dst_v, src_v, u_v, v_v, ostg, s_acc):
    c = lax.axis_index("c")
    s = lax.axis_index("s")
    wid = s * 2 + c

    for t in range((NR * SUB + 31) // 32):
        blk = wid + t * 32

        @pl.when(blk < NR * SUB)
        def _():
            r = blk // SUB
            pltpu.sync_copy(ds.at[blk], dst_v)
            pltpu.sync_copy(sr.at[blk], src_v)
            pltpu.sync_copy(u5.at[r], u_v)
            pltpu.sync_copy(v5.at[r], v_v)
            roff = r * N

            def sz_body(j, carry):
                s_acc[pl.ds(j * 16, 16)] = jnp.zeros((16,), _f32)
                return carry

            lax.fori_loop(0, N // 16, sz_body, 0)

            def p_body(j, carry):
                for k in range(B // 16):
                    sl = pl.ds(k * 16, 16)
                    d16 = dst_v[j, sl]
                    s16 = src_v[j, sl]
                    ud = plsc.load_gather(u_v, [d16])
                    vs = plsc.load_gather(v_v, [s16])
                    p16 = jnp.exp(ud + vs)
                    plsc.addupdate_scatter(s_acc, [d16], p16)
                    ostg[j, 0, sl] = d16
                    ostg[j, 1, sl] = s16 + roff
                    ostg[j, 2, sl] = plsc.bitcast(p16, _i32)
                return carry

            lax.fori_loop(0, NCH, p_body, 0)
            pltpu.sync_copy(ostg, edata.at[pl.ds(blk * NCH, NCH)])
            pltpu.sync_copy(s_acc, spart.at[blk])


@functools.partial(
    pl.kernel,
    out_type=jax.ShapeDtypeStruct((NR * 2 * N, HW), _f32),
    mesh=_sc_mesh,
    compiler_params=_sc_params,
    scratch_types=[
        pltpu.VMEM((3, B), _i32),
        pltpu.VMEM((3, B), _i32),
        pltpu.VMEM((3, B), _i32),
        pltpu.VMEM((3, B), _i32),
        pltpu.VMEM((B, HW), _f32),
        pltpu.VMEM((B, HW), _f32),
        pltpu.VMEM((B, HW), _f32),
        pltpu.VMEM((B, HW), _f32),
        pltpu.VMEM_SHARED((N, HW), _f32),
        pltpu.SemaphoreType.DMA,
        pltpu.SemaphoreType.DMA,
        pltpu.SemaphoreType.DMA,
        pltpu.SemaphoreType.DMA,
        pltpu.SemaphoreType.DMA,
        pltpu.SemaphoreType.DMA,
        pltpu.SemaphoreType.DMA,
        pltpu.SemaphoreType.DMA,
        pltpu.SemaphoreType.DMA,
        pltpu.SemaphoreType.DMA,
        pltpu.SemaphoreType.DMA,
        pltpu.SemaphoreType.DMA,
    ],
)
def _sc_acc(ft0, ft1, edata, out,
            eb0, eb1, eb2, eb3, rw0, rw1, rw2, rw3, acc,
            es0, es1, es2, es3, gs0, gs1, gs2, gs3, ss0, ss1, ss2, ss3):
    c = lax.axis_index("c")
    s = lax.axis_index("s")
    ebuf = (eb0, eb1, eb2, eb3)
    esem = (es0, es1, es2, es3)
    rows = (rw0, rw1, rw2, rw3)
    gsem = (gs0, gs1, gs2, gs3)
    ssem = (ss0, ss1, ss2, ss3)
    nblk_n = N // B

    def rel_body(r, carry):
        base = (r * SUB + s) * NCH

        def z_body(rr, carry1):
            for k in range(HW // 16):
                rw0[rr, pl.ds(k * 16, 16)] = jnp.zeros((16,), _f32)
            return carry1

        lax.fori_loop(0, B, z_body, 0)
        for t in range((nblk_n + SUB - 1) // SUB):
            blk = s + t * SUB

            @pl.when(blk < nblk_n)
            def _():
                pltpu.sync_copy(rw0, acc.at[pl.ds(blk * B, B)])
        plsc.subcore_barrier()

        pltpu.async_copy(edata.at[base], eb0, es0)
        pltpu.async_copy(edata.at[base + 1], eb1, es1)

        def m_body(m, carry2):
            for q in range(4):
                j = 4 * m + q
                q2 = (q + 2) % 4
                qp = (q + 3) % 4

                @pl.when(j + 2 < NCH)
                def _():
                    @pl.when(j >= 2)
                    def _():
                        pltpu.make_async_copy(
                            rows[q2], acc.at[ebuf[q2].at[0]],
                            ssem[q2]).wait()
                    pltpu.async_copy(edata.at[base + j + 2],
                                     ebuf[q2], esem[q2])

                @pl.when(j < NCH)
                def _():
                    pltpu.make_async_copy(
                        edata.at[base], ebuf[q], esem[q]).wait()

                    @pl.when(c == 0)
                    def _():
                        pltpu.async_copy(ft0.at[ebuf[q].at[1]], rows[q],
                                         gsem[q])

                    @pl.when(c == 1)
                    def _():
                        pltpu.async_copy(ft1.at[ebuf[q].at[1]], rows[q],
                                         gsem[q])

                @pl.when((j >= 1) & (j <= NCH))
                def _():
                    pltpu.make_async_copy(
                        ft0.at[ebuf[qp].at[1]], rows[qp], gsem[qp]).wait()

                    def scale_grp(g, carry3):
                        pvec = plsc.bitcast(
                            ebuf[qp][2, pl.ds(g * 16, 16)], _f32)
                        for rr in range(16):
                            rowi = g * 16 + rr
                            spl = pvec[rr]
                            for k in range(HW // 16):
                                sl = pl.ds(k * 16, 16)
                                rows[qp][rowi, sl] = (
                                    rows[qp][rowi, sl] * spl)
                        return carry3

                    lax.fori_loop(0, B // 16, scale_grp, 0)
                    pltpu.async_copy(rows[qp], acc.at[ebuf[qp].at[0]],
                                     ssem[qp], add=True)
            return carry2

        lax.fori_loop(0, (NCH + 2 + 3) // 4, m_body, 0)
        for q in range(4):
            pltpu.make_async_copy(rows[q], acc.at[eb0.at[0]],
                                  ssem[q]).wait()
        plsc.subcore_barrier()

        for t in range((nblk_n + SUB - 1) // SUB):
            blk = s + t * SUB

            @pl.when(blk < nblk_n)
            def _():
                pltpu.sync_copy(acc.at[pl.ds(blk * B, B)], rw0)
                pltpu.sync_copy(
                    rw0, out.at[pl.ds((2 * r + c) * N + blk * B, B)])
        return carry

    lax.fori_loop(0, NR, rel_body, 0)


def kernel(concept_emb, item_emb, stu_emb, cc_edge_index, ic_edge_index,
           si_edge_index, Wc1, Wc2, Wi1, Wi2, Ws,
           ac1_w, ac1_b, ac2_w, ac2_b, ace1_w, ace1_b, ace2_w, ace2_b,
           ai1_w, ai1_b, ai2_w, ai2_b, aie1_w, aie1_b, aie2_w, aie2_b,
           asl_w, asl_b):
    ce, ie, se = concept_emb, item_emb, stu_emb
    cc, ic, si = cc_edge_index, ic_edge_index, si_edge_index

    rels = [
        (ce, ce, Wc1, ac1_w, cc[1], cc[0]),
        (ce, ie, Wc2, ac2_w, ic[0], ic[1]),
        (ie, ce, Wi1, ai1_w, ic[1], ic[0]),
        (ie, se, Wi2, ai2_w, si[0], si[1]),
        (se, ie, Ws, asl_w, si[1], si[0]),
    ]

    u0s, vs, f0s, f1s, dss, srs = [], [], [], [], [], []
    for self_t, nbr_t, W, aw, src, dst in rels:
        f0, f1, v, u0 = _dense(nbr_t, self_t, W,
                               aw[Dm:].reshape(Dm, 1), aw[:Dm].reshape(Dm, 1))
        u0s.append(u0.reshape(N))
        vs.append(v.reshape(N))
        f0s.append(f0)
        f1s.append(f1)
        dss.append(dst.reshape(SUB, NCH, B))
        srs.append(src.reshape(SUB, NCH, B))

    v5 = jnp.stack(vs)
    u5 = _shift(jnp.stack(u0s).reshape(NR, 1, N),
                v5.reshape(NR, 1, N)).reshape(NR, N)
    ft0 = jnp.concatenate(f0s, axis=0)
    ft1 = jnp.concatenate(f1s, axis=0)
    ds = jnp.concatenate(dss, axis=0)
    sr = jnp.concatenate(srs, axis=0)

    edata, spart = _sc_pw(u5, v5, ds, sr)
    o = _sc_acc(ft0, ft1, edata)

    o = o.reshape(NR, 2, N, HW)
    sp = spart.reshape(NR, SUB, N).transpose(0, 2, 1)

    conc_out, item_out, stu_out = _combine(
        ce, o[0, 0], o[0, 1], sp[0], o[1, 0], o[1, 1], sp[1],
        ie, o[2, 0], o[2, 1], sp[2], o[3, 0], o[3, 1], sp[3],
        se, o[4, 0], o[4, 1], sp[4])
    return (conc_out, item_out, stu_out)
